# Initial kernel scaffold; baseline (speedup 1.0000x reference)
import jax
import jax.numpy as jnp
from jax.experimental import pallas as pl


def kernel(x, edge_index, edge_attr, target_edges, W1, b1, W2, b2, W3, b3, M1w, M1b, M2w, M2b, M3w, M3b):
    raise NotImplementedError("write your pallas kernel here")


# speedup vs baseline: 7.5164x; 7.5164x over previous
import functools

import jax
import jax.numpy as jnp
from jax import lax
from jax.experimental import pallas as pl
from jax.experimental.pallas import tpu as pltpu
from jax.experimental.pallas import tpu_sc as plsc

_N = 10000
_E = 320000
_T = 100000
_F = 128
_DE = 16
_H = 64

_NC, _NS, _L = 2, 16, 16
_NW = _NC * _NS

_NP = 10016
_ROWS_T = _NP // _NS
_CH = 128
_ECH = 80
_EPT = _CH * _ECH
_EP = _EPT * _NW
_TCH = 25
_TPT = _CH * _TCH
_TP = _TPT * _NW
_DCH = 2048

_f32 = jnp.float32
_i32 = jnp.int32


@functools.cache
def _sc_kernels():
  mesh = plsc.VectorSubcoreMesh(core_axis_name="c", subcore_axis_name="s",
                                num_cores=_NC, num_subcores=_NS)

  @functools.partial(
      pl.kernel,
      out_type=(
          jax.ShapeDtypeStruct((_NW, _NP), _f32),
          jax.ShapeDtypeStruct((_TP,), _i32),
          jax.ShapeDtypeStruct((_TP,), _i32),
          jax.ShapeDtypeStruct((_TP, _DE), _f32),
      ),
      mesh=mesh,
      compiler_params=pltpu.CompilerParams(needs_layout_passes=False, use_tc_tiling_on_sc=False),
      scratch_types=[
          pltpu.VMEM((_NP,), _f32),
          pltpu.VMEM((_DCH,), _i32),
          pltpu.VMEM((_CH,), _i32),
          pltpu.VMEM((_CH,), _i32),
          pltpu.VMEM((_CH,), _i32),
          pltpu.VMEM((_CH, _DE), _f32),
          pltpu.SemaphoreType.DMA,
      ],
  )
  def sc_prep(dst_hbm, te_hbm, ei0_hbm, ei1_hbm, ea_hbm,
              degp_hbm, sidx_hbm, didx_hbm, ag_hbm,
              degloc, idxbuf, tev, sv, dv, arow, sem):
    c = lax.axis_index("c")
    s = lax.axis_index("s")
    w = s * _NC + c
    ones16 = jnp.full((_L,), 1.0, _f32)
    zer16 = jnp.zeros((_L,), _f32)

    @pl.loop(0, _NP // _L)
    def _(i):
      degloc[pl.ds(i * _L, _L)] = zer16

    ebase = w * _EPT

    @pl.loop(0, _EPT // _DCH)
    def _(jc):
      pltpu.sync_copy(dst_hbm.at[pl.ds(ebase + jc * _DCH, _DCH)], idxbuf)

      @pl.loop(0, _DCH // _L)
      def _(i):
        plsc.addupdate_scatter(degloc, [idxbuf[pl.ds(i * _L, _L)]], ones16)

    pltpu.sync_copy(degloc, degp_hbm.at[w])

    tbase = w * _TPT

    @pl.loop(0, _TCH)
    def _(j):
      t0 = tbase + j * _CH
      pltpu.sync_copy(te_hbm.at[pl.ds(t0, _CH)], tev)
      pltpu.async_copy(ei0_hbm.at[tev], sv, sem).wait()
      pltpu.async_copy(ei1_hbm.at[tev], dv, sem).wait()
      pltpu.async_copy(ea_hbm.at[tev], arow, sem).wait()
      pltpu.sync_copy(sv, sidx_hbm.at[pl.ds(t0, _CH)])
      pltpu.sync_copy(dv, didx_hbm.at[pl.ds(t0, _CH)])
      pltpu.sync_copy(arow, ag_hbm.at[pl.ds(t0, _CH)])

  @functools.partial(
      pl.kernel,
      out_type=jax.ShapeDtypeStruct((_NC, _NP, _H), _f32),
      mesh=mesh,
      compiler_params=pltpu.CompilerParams(needs_layout_passes=False, use_tc_tiling_on_sc=False),
      scratch_types=[
          pltpu.VMEM((_CH,), _i32),
          pltpu.VMEM((_CH,), _i32),
          pltpu.VMEM((_CH, _H), _f32),
          pltpu.VMEM_SHARED((_NP, _H), _f32),
          pltpu.SemaphoreType.DMA,
      ],
  )
  def sc_scatter(src_hbm, dst_hbm, g_hbm, z_hbm, out_hbm,
                 idxs, idxd, rows, acc, sem):
    c = lax.axis_index("c")
    s = lax.axis_index("s")
    row0 = s * _ROWS_T

    @pl.when(c == 0)
    def _():
      pltpu.sync_copy(g_hbm.at[pl.ds(row0, _ROWS_T)],
                      acc.at[pl.ds(row0, _ROWS_T)])

    @pl.when(c == 1)
    def _():
      pltpu.sync_copy(z_hbm.at[pl.ds(row0, _ROWS_T)],
                      acc.at[pl.ds(row0, _ROWS_T)])

    plsc.subcore_barrier()
    base = (c * _NS + s) * _EPT

    @pl.loop(0, _ECH)
    def _(j):
      e0 = base + j * _CH
      pltpu.sync_copy(src_hbm.at[pl.ds(e0, _CH)], idxs)
      pltpu.sync_copy(dst_hbm.at[pl.ds(e0, _CH)], idxd)
      pltpu.async_copy(g_hbm.at[idxs], rows, sem).wait()
      pltpu.sync_copy(rows, acc.at[idxd], add=True)

    plsc.subcore_barrier()
    pltpu.sync_copy(acc.at[pl.ds(row0, _ROWS_T)],
                    out_hbm.at[c, pl.ds(row0, _ROWS_T)])

  @functools.partial(
      pl.kernel,
      out_type=(
          jax.ShapeDtypeStruct((_TP, _H), _f32),
          jax.ShapeDtypeStruct((_TP, _H), _f32),
      ),
      mesh=mesh,
      compiler_params=pltpu.CompilerParams(needs_layout_passes=False, use_tc_tiling_on_sc=False),
      scratch_types=[
          pltpu.VMEM((_CH,), _i32),
          pltpu.VMEM((_CH,), _i32),
          pltpu.VMEM((_CH, _H), _f32),
          pltpu.VMEM((_CH, _H), _f32),
          pltpu.SemaphoreType.DMA,
      ],
  )
  def sc_pq(sidx_hbm, didx_hbm, p_hbm, q_hbm, pg_hbm, qg_hbm,
            sv, dv, prow, qrow, sem):
    c = lax.axis_index("c")
    s = lax.axis_index("s")
    w = s * _NC + c
    tbase = w * _TPT

    @pl.loop(0, _TCH)
    def _(j):
      t0 = tbase + j * _CH
      pltpu.sync_copy(sidx_hbm.at[pl.ds(t0, _CH)], sv)
      pltpu.sync_copy(didx_hbm.at[pl.ds(t0, _CH)], dv)
      pltpu.async_copy(p_hbm.at[sv], prow, sem).wait()
      pltpu.async_copy(q_hbm.at[dv], qrow, sem).wait()
      pltpu.sync_copy(prow, pg_hbm.at[pl.ds(t0, _CH)])
      pltpu.sync_copy(qrow, qg_hbm.at[pl.ds(t0, _CH)])

  return sc_prep, sc_scatter, sc_pq


def _tc1_body(degp_ref, x_ref, w_ref, dinv_ref, g_ref):
  deg = jnp.sum(degp_ref[...], axis=1, keepdims=True) + 1.0
  dinv = lax.rsqrt(deg)
  dinv_ref[...] = dinv
  pre = jnp.dot(x_ref[...], w_ref[...], preferred_element_type=_f32)
  g_ref[...] = pre * dinv


_tc1 = pl.pallas_call(
    _tc1_body,
    out_shape=(
        jax.ShapeDtypeStruct((_NP, 1), _f32),
        jax.ShapeDtypeStruct((_NP, _H), _f32),
    ),
)


def _comb_body(p_ref, dinv_ref, b_ref, w_ref, g_ref):
  dinv = dinv_ref[...]
  h = jnp.maximum((p_ref[0] + p_ref[1]) * dinv + b_ref[...], 0.0)
  pre = jnp.dot(h, w_ref[...], preferred_element_type=_f32)
  g_ref[...] = pre * dinv


_comb = pl.pallas_call(
    _comb_body,
    out_shape=jax.ShapeDtypeStruct((_NP, _H), _f32),
)


def _comb3_body(p_ref, dinv_ref, b_ref, wa_ref, wb_ref, pout_ref, qout_ref):
  dinv = dinv_ref[...]
  h = jnp.maximum((p_ref[0] + p_ref[1]) * dinv + b_ref[...], 0.0)
  pout_ref[...] = jnp.dot(h, wa_ref[...], preferred_element_type=_f32)
  qout_ref[...] = jnp.dot(h, wb_ref[...], preferred_element_type=_f32)


_comb3 = pl.pallas_call(
    _comb3_body,
    out_shape=(
        jax.ShapeDtypeStruct((_NP, _H), _f32),
        jax.ShapeDtypeStruct((_NP, _H), _f32),
    ),
)


def _mlp_body(pg_ref, qg_ref, ag_ref, m1c_ref, m1b_ref, m2w_ref, m2b_ref,
              m3w_ref, m3b_ref, o_ref):
  e1 = (pg_ref[...] + qg_ref[...]
        + jnp.dot(ag_ref[...], m1c_ref[...], preferred_element_type=_f32)
        + m1b_ref[...])
  e1 = jnp.maximum(e1, 0.0)
  e2 = jnp.maximum(
      jnp.dot(e1, m2w_ref[...], preferred_element_type=_f32) + m2b_ref[...],
      0.0)
  z = jnp.sum(e2 * m3w_ref[...], axis=1, keepdims=True) + m3b_ref[...]
  o_ref[...] = 1.0 / (1.0 + jnp.exp(-z))


_MLP_GRID = 16
_MR = _TP // _MLP_GRID

_mlp = pl.pallas_call(
    _mlp_body,
    grid=(_MLP_GRID,),
    in_specs=[
        pl.BlockSpec((_MR, _H), lambda i: (i, 0)),
        pl.BlockSpec((_MR, _H), lambda i: (i, 0)),
        pl.BlockSpec((_MR, _DE), lambda i: (i, 0)),
        pl.BlockSpec((_DE, _H), lambda i: (0, 0)),
        pl.BlockSpec((1, _H), lambda i: (0, 0)),
        pl.BlockSpec((_H, _H // 2), lambda i: (0, 0)),
        pl.BlockSpec((1, _H // 2), lambda i: (0, 0)),
        pl.BlockSpec((1, _H // 2), lambda i: (0, 0)),
        pl.BlockSpec((1, 1), lambda i: (0, 0)),
    ],
    out_specs=pl.BlockSpec((_MR, 1), lambda i: (i, 0)),
    out_shape=jax.ShapeDtypeStruct((_TP, 1), _f32),
)


def kernel(x, edge_index, edge_attr, target_edges,
           W1, b1, W2, b2, W3, b3, M1w, M1b, M2w, M2b, M3w, M3b):
  sc_prep, sc_scatter, sc_pq = _sc_kernels()
  src = edge_index[0]
  dst = edge_index[1]
  pad_e = jnp.full((_EP - _E,), _NP - 1, dtype=_i32)
  src_p = jnp.concatenate([src, pad_e])
  dst_p = jnp.concatenate([dst, pad_e])
  te_p = jnp.concatenate([target_edges, jnp.zeros((_TP - _T,), _i32)])
  x_p = jnp.pad(x, ((0, _NP - _N), (0, 0)))
  zeros_nh = jnp.zeros((_NP, _H), _f32)

  degp, sidx, didx, ag = sc_prep(dst_p, te_p, src, dst, edge_attr)
  dinv, g1 = _tc1(degp.T, x_p, W1)
  p = sc_scatter(src_p, dst_p, g1, zeros_nh)
  g2 = _comb(p, dinv, b1.reshape(1, _H), W2)
  p = sc_scatter(src_p, dst_p, g2, zeros_nh)
  g3 = _comb(p, dinv, b2.reshape(1, _H), W3)
  p = sc_scatter(src_p, dst_p, g3, zeros_nh)
  P, Q = _comb3(p, dinv, b3.reshape(1, _H), M1w[:_H], M1w[_H:2 * _H])
  pg, qg = sc_pq(sidx, didx, P, Q)
  out = _mlp(pg, qg, ag, M1w[2 * _H:], M1b.reshape(1, _H),
             M2w, M2b.reshape(1, _H // 2), M3w.reshape(1, _H // 2),
             M3b.reshape(1, 1))
  return out[:_T, 0]


# speedup vs baseline: 9.3444x; 1.2432x over previous
import functools

import jax
import jax.numpy as jnp
from jax import lax
from jax.experimental import pallas as pl
from jax.experimental.pallas import tpu as pltpu
from jax.experimental.pallas import tpu_sc as plsc

_N = 10000
_E = 320000
_T = 100000
_F = 128
_DE = 16
_H = 64

_NC, _NS, _L = 2, 16, 16
_NW = _NC * _NS

_NP = 10016
_ROWS_T = _NP // _NS
_CH = 128
_ECH = 80
_EPT = _CH * _ECH
_EP = _EPT * _NW
_TCH = 25
_TPT = _CH * _TCH
_TP = _TPT * _NW
_DCH = 2048

_f32 = jnp.float32
_i32 = jnp.int32


@functools.cache
def _sc_kernels():
  mesh = plsc.VectorSubcoreMesh(core_axis_name="c", subcore_axis_name="s",
                                num_cores=_NC, num_subcores=_NS)

  @functools.partial(
      pl.kernel,
      out_type=(
          jax.ShapeDtypeStruct((_NW, _NP), _f32),
          jax.ShapeDtypeStruct((_TP,), _i32),
          jax.ShapeDtypeStruct((_TP,), _i32),
          jax.ShapeDtypeStruct((_TP, _DE), _f32),
      ),
      mesh=mesh,
      compiler_params=pltpu.CompilerParams(needs_layout_passes=False, use_tc_tiling_on_sc=False),
      scratch_types=[
          pltpu.VMEM((_NP,), _f32),
          pltpu.VMEM((_DCH,), _i32),
          pltpu.VMEM((_CH,), _i32),
          pltpu.VMEM((_CH,), _i32),
          pltpu.VMEM((_CH,), _i32),
          pltpu.VMEM((_CH, _DE), _f32),
          pltpu.SemaphoreType.DMA,
      ],
  )
  def sc_prep(dst_hbm, te_hbm, ei0_hbm, ei1_hbm, ea_hbm,
              degp_hbm, sidx_hbm, didx_hbm, ag_hbm,
              degloc, idxbuf, tev, sv, dv, arow, sem):
    c = lax.axis_index("c")
    s = lax.axis_index("s")
    w = s * _NC + c
    ones16 = jnp.full((_L,), 1.0, _f32)
    zer16 = jnp.zeros((_L,), _f32)

    @pl.loop(0, _NP // _L)
    def _(i):
      degloc[pl.ds(i * _L, _L)] = zer16

    ebase = w * _EPT

    @pl.loop(0, _EPT // _DCH)
    def _(jc):
      pltpu.sync_copy(dst_hbm.at[pl.ds(ebase + jc * _DCH, _DCH)], idxbuf)

      @pl.loop(0, _DCH // _L)
      def _(i):
        plsc.addupdate_scatter(degloc, [idxbuf[pl.ds(i * _L, _L)]], ones16)

    pltpu.sync_copy(degloc, degp_hbm.at[w])

    tbase = w * _TPT

    @pl.loop(0, _TCH)
    def _(j):
      t0 = tbase + j * _CH
      pltpu.sync_copy(te_hbm.at[pl.ds(t0, _CH)], tev)
      pltpu.async_copy(ei0_hbm.at[tev], sv, sem).wait()
      pltpu.async_copy(ei1_hbm.at[tev], dv, sem).wait()
      pltpu.async_copy(ea_hbm.at[tev], arow, sem).wait()
      pltpu.sync_copy(sv, sidx_hbm.at[pl.ds(t0, _CH)])
      pltpu.sync_copy(dv, didx_hbm.at[pl.ds(t0, _CH)])
      pltpu.sync_copy(arow, ag_hbm.at[pl.ds(t0, _CH)])

  @functools.partial(
      pl.kernel,
      out_type=jax.ShapeDtypeStruct((_NC, _NP, _H), _f32),
      mesh=mesh,
      compiler_params=pltpu.CompilerParams(needs_layout_passes=False, use_tc_tiling_on_sc=False),
      scratch_types=[
          pltpu.VMEM((_ECH, _CH), _i32),
          pltpu.VMEM((_ECH, _CH), _i32),
          pltpu.VMEM((_CH, _H), _f32),
          pltpu.VMEM((_CH, _H), _f32),
          pltpu.VMEM_SHARED((_NP, _H), _f32),
          pltpu.SemaphoreType.DMA,
          pltpu.SemaphoreType.DMA,
      ],
  )
  def sc_scatter(src2_hbm, dst2_hbm, g_hbm, z_hbm, out_hbm,
                 idxs2, idxd2, rows0, rows1, acc, sem0, sem1):
    c = lax.axis_index("c")
    s = lax.axis_index("s")
    row0 = s * _ROWS_T

    chunk0 = (c * _NS + s) * _ECH
    pltpu.sync_copy(src2_hbm.at[pl.ds(chunk0, _ECH)], idxs2)
    pltpu.sync_copy(dst2_hbm.at[pl.ds(chunk0, _ECH)], idxd2)

    @pl.when(c == 0)
    def _():
      pltpu.sync_copy(g_hbm.at[pl.ds(row0, _ROWS_T)],
                      acc.at[pl.ds(row0, _ROWS_T)])

    @pl.when(c == 1)
    def _():
      pltpu.sync_copy(z_hbm.at[pl.ds(row0, _ROWS_T)],
                      acc.at[pl.ds(row0, _ROWS_T)])

    pltpu.async_copy(g_hbm.at[idxs2.at[0]], rows0, sem0)
    pltpu.async_copy(g_hbm.at[idxs2.at[1]], rows1, sem1)
    plsc.subcore_barrier()

    @pl.loop(0, _ECH, step=2)
    def _(j):
      pltpu.make_async_copy(g_hbm.at[idxs2.at[0]], rows0, sem0).wait()
      pltpu.sync_copy(rows0, acc.at[idxd2.at[j]], add=True)

      @pl.when(j + 2 < _ECH)
      def _():
        pltpu.async_copy(g_hbm.at[idxs2.at[j + 2]], rows0, sem0)

      pltpu.make_async_copy(g_hbm.at[idxs2.at[1]], rows1, sem1).wait()
      pltpu.sync_copy(rows1, acc.at[idxd2.at[j + 1]], add=True)

      @pl.when(j + 3 < _ECH)
      def _():
        pltpu.async_copy(g_hbm.at[idxs2.at[j + 3]], rows1, sem1)

    plsc.subcore_barrier()
    pltpu.sync_copy(acc.at[pl.ds(row0, _ROWS_T)],
                    out_hbm.at[c, pl.ds(row0, _ROWS_T)])

  @functools.partial(
      pl.kernel,
      out_type=(
          jax.ShapeDtypeStruct((_TP, _H), _f32),
          jax.ShapeDtypeStruct((_TP, _H), _f32),
      ),
      mesh=mesh,
      compiler_params=pltpu.CompilerParams(needs_layout_passes=False, use_tc_tiling_on_sc=False),
      scratch_types=[
          pltpu.VMEM((_CH,), _i32),
          pltpu.VMEM((_CH,), _i32),
          pltpu.VMEM((_CH, _H), _f32),
          pltpu.VMEM((_CH, _H), _f32),
          pltpu.SemaphoreType.DMA,
      ],
  )
  def sc_pq(sidx_hbm, didx_hbm, p_hbm, q_hbm, pg_hbm, qg_hbm,
            sv, dv, prow, qrow, sem):
    c = lax.axis_index("c")
    s = lax.axis_index("s")
    w = s * _NC + c
    tbase = w * _TPT

    @pl.loop(0, _TCH)
    def _(j):
      t0 = tbase + j * _CH
      pltpu.sync_copy(sidx_hbm.at[pl.ds(t0, _CH)], sv)
      pltpu.sync_copy(didx_hbm.at[pl.ds(t0, _CH)], dv)
      pltpu.async_copy(p_hbm.at[sv], prow, sem).wait()
      pltpu.async_copy(q_hbm.at[dv], qrow, sem).wait()
      pltpu.sync_copy(prow, pg_hbm.at[pl.ds(t0, _CH)])
      pltpu.sync_copy(qrow, qg_hbm.at[pl.ds(t0, _CH)])

  return sc_prep, sc_scatter, sc_pq


def _tc1_body(degp_ref, x_ref, w_ref, dinv_ref, g_ref):
  deg = jnp.sum(degp_ref[...], axis=1, keepdims=True) + 1.0
  dinv = lax.rsqrt(deg)
  dinv_ref[...] = dinv
  pre = jnp.dot(x_ref[...], w_ref[...], preferred_element_type=_f32)
  g_ref[...] = pre * dinv


_tc1 = pl.pallas_call(
    _tc1_body,
    out_shape=(
        jax.ShapeDtypeStruct((_NP, 1), _f32),
        jax.ShapeDtypeStruct((_NP, _H), _f32),
    ),
)


def _comb_body(p_ref, dinv_ref, b_ref, w_ref, g_ref):
  dinv = dinv_ref[...]
  h = jnp.maximum((p_ref[0] + p_ref[1]) * dinv + b_ref[...], 0.0)
  pre = jnp.dot(h, w_ref[...], preferred_element_type=_f32)
  g_ref[...] = pre * dinv


_comb = pl.pallas_call(
    _comb_body,
    out_shape=jax.ShapeDtypeStruct((_NP, _H), _f32),
)


def _comb3_body(p_ref, dinv_ref, b_ref, wa_ref, wb_ref, pout_ref, qout_ref):
  dinv = dinv_ref[...]
  h = jnp.maximum((p_ref[0] + p_ref[1]) * dinv + b_ref[...], 0.0)
  pout_ref[...] = jnp.dot(h, wa_ref[...], preferred_element_type=_f32)
  qout_ref[...] = jnp.dot(h, wb_ref[...], preferred_element_type=_f32)


_comb3 = pl.pallas_call(
    _comb3_body,
    out_shape=(
        jax.ShapeDtypeStruct((_NP, _H), _f32),
        jax.ShapeDtypeStruct((_NP, _H), _f32),
    ),
)


def _mlp_body(pg_ref, qg_ref, ag_ref, m1c_ref, m1b_ref, m2w_ref, m2b_ref,
              m3w_ref, m3b_ref, o_ref):
  e1 = (pg_ref[...] + qg_ref[...]
        + jnp.dot(ag_ref[...], m1c_ref[...], preferred_element_type=_f32)
        + m1b_ref[...])
  e1 = jnp.maximum(e1, 0.0)
  e2 = jnp.maximum(
      jnp.dot(e1, m2w_ref[...], preferred_element_type=_f32) + m2b_ref[...],
      0.0)
  z = jnp.sum(e2 * m3w_ref[...], axis=1, keepdims=True) + m3b_ref[...]
  o_ref[...] = 1.0 / (1.0 + jnp.exp(-z))


_MLP_GRID = 16
_MR = _TP // _MLP_GRID

_mlp = pl.pallas_call(
    _mlp_body,
    grid=(_MLP_GRID,),
    in_specs=[
        pl.BlockSpec((_MR, _H), lambda i: (i, 0)),
        pl.BlockSpec((_MR, _H), lambda i: (i, 0)),
        pl.BlockSpec((_MR, _DE), lambda i: (i, 0)),
        pl.BlockSpec((_DE, _H), lambda i: (0, 0)),
        pl.BlockSpec((1, _H), lambda i: (0, 0)),
        pl.BlockSpec((_H, _H // 2), lambda i: (0, 0)),
        pl.BlockSpec((1, _H // 2), lambda i: (0, 0)),
        pl.BlockSpec((1, _H // 2), lambda i: (0, 0)),
        pl.BlockSpec((1, 1), lambda i: (0, 0)),
    ],
    out_specs=pl.BlockSpec((_MR, 1), lambda i: (i, 0)),
    out_shape=jax.ShapeDtypeStruct((_TP, 1), _f32),
)


def kernel(x, edge_index, edge_attr, target_edges,
           W1, b1, W2, b2, W3, b3, M1w, M1b, M2w, M2b, M3w, M3b):
  sc_prep, sc_scatter, sc_pq = _sc_kernels()
  src = edge_index[0]
  dst = edge_index[1]
  pad_e = jnp.full((_EP - _E,), _NP - 1, dtype=_i32)
  src_p = jnp.concatenate([src, pad_e])
  dst_p = jnp.concatenate([dst, pad_e])
  te_p = jnp.concatenate([target_edges, jnp.zeros((_TP - _T,), _i32)])
  x_p = jnp.pad(x, ((0, _NP - _N), (0, 0)))
  zeros_nh = jnp.zeros((_NP, _H), _f32)

  src2 = src_p.reshape(_EP // _CH, _CH)
  dst2 = dst_p.reshape(_EP // _CH, _CH)

  degp, sidx, didx, ag = sc_prep(dst_p, te_p, src, dst, edge_attr)
  dinv, g1 = _tc1(degp.T, x_p, W1)
  p = sc_scatter(src2, dst2, g1, zeros_nh)
  g2 = _comb(p, dinv, b1.reshape(1, _H), W2)
  p = sc_scatter(src2, dst2, g2, zeros_nh)
  g3 = _comb(p, dinv, b2.reshape(1, _H), W3)
  p = sc_scatter(src2, dst2, g3, zeros_nh)
  P, Q = _comb3(p, dinv, b3.reshape(1, _H), M1w[:_H], M1w[_H:2 * _H])
  pg, qg = sc_pq(sidx, didx, P, Q)
  out = _mlp(pg, qg, ag, M1w[2 * _H:], M1b.reshape(1, _H),
             M2w, M2b.reshape(1, _H // 2), M3w.reshape(1, _H // 2),
             M3b.reshape(1, 1))
  return out[:_T, 0]


# speedup vs baseline: 10.7538x; 1.1508x over previous
import functools

import jax
import jax.numpy as jnp
from jax import lax
from jax.experimental import pallas as pl
from jax.experimental.pallas import tpu as pltpu
from jax.experimental.pallas import tpu_sc as plsc

_N = 10000
_E = 320000
_T = 100000
_F = 128
_DE = 16
_H = 64

_NC, _NS, _L = 2, 16, 16
_NW = _NC * _NS

_NP = 10016
_ROWS_T = _NP // _NS
_CH = 128
_ECH = 80
_EPT = _CH * _ECH
_EP = _EPT * _NW
_TCH = 25
_TPT = _CH * _TCH
_TP = _TPT * _NW
_DCH = 2048

_f32 = jnp.float32
_i32 = jnp.int32


@functools.cache
def _sc_kernels():
  mesh = plsc.VectorSubcoreMesh(core_axis_name="c", subcore_axis_name="s",
                                num_cores=_NC, num_subcores=_NS)

  @functools.partial(
      pl.kernel,
      out_type=(
          jax.ShapeDtypeStruct((_NW, _NP), _f32),
          jax.ShapeDtypeStruct((_TP,), _i32),
          jax.ShapeDtypeStruct((_TP,), _i32),
          jax.ShapeDtypeStruct((_TP, _DE), _f32),
      ),
      mesh=mesh,
      compiler_params=pltpu.CompilerParams(needs_layout_passes=False, use_tc_tiling_on_sc=False),
      scratch_types=[
          pltpu.VMEM((_NP,), _f32),
          pltpu.VMEM((_DCH,), _i32),
          pltpu.VMEM((_TCH, _CH), _i32),
          pltpu.VMEM((_CH,), _i32),
          pltpu.VMEM((_CH,), _i32),
          pltpu.VMEM((_CH, _DE), _f32),
          pltpu.VMEM((_CH,), _i32),
          pltpu.VMEM((_CH,), _i32),
          pltpu.VMEM((_CH, _DE), _f32),
          pltpu.SemaphoreType.DMA,
          pltpu.SemaphoreType.DMA,
      ],
  )
  def sc_prep(dst_hbm, te2_hbm, ei0_hbm, ei1_hbm, ea_hbm,
              degp_hbm, sidx_hbm, didx_hbm, ag_hbm,
              degloc, idxbuf, te2, sv0, dv0, ar0, sv1, dv1, ar1,
              sem0, sem1):
    c = lax.axis_index("c")
    s = lax.axis_index("s")
    w = s * _NC + c
    ones16 = jnp.full((_L,), 1.0, _f32)
    zer16 = jnp.zeros((_L,), _f32)

    tchunk0 = w * _TCH
    pltpu.sync_copy(te2_hbm.at[pl.ds(tchunk0, _TCH)], te2)

    def tgt_start(j, sv, dv, ar, sem):
      pltpu.async_copy(ei0_hbm.at[te2.at[j]], sv, sem)
      pltpu.async_copy(ei1_hbm.at[te2.at[j]], dv, sem)
      pltpu.async_copy(ea_hbm.at[te2.at[j]], ar, sem)

    def tgt_finish(j, sv, dv, ar, sem):
      t0 = w * _TPT + j * _CH
      pltpu.make_async_copy(ei0_hbm.at[te2.at[0]], sv, sem).wait()
      pltpu.make_async_copy(ei1_hbm.at[te2.at[0]], dv, sem).wait()
      pltpu.make_async_copy(ea_hbm.at[te2.at[0]], ar, sem).wait()
      pltpu.sync_copy(sv, sidx_hbm.at[pl.ds(t0, _CH)])
      pltpu.sync_copy(dv, didx_hbm.at[pl.ds(t0, _CH)])
      pltpu.sync_copy(ar, ag_hbm.at[pl.ds(t0, _CH)])

    tgt_start(0, sv0, dv0, ar0, sem0)
    tgt_start(1, sv1, dv1, ar1, sem1)

    @pl.loop(0, _NP // _L)
    def _(i):
      degloc[pl.ds(i * _L, _L)] = zer16

    ebase = w * _EPT

    @pl.loop(0, _EPT // _DCH)
    def _(jc):
      pltpu.sync_copy(dst_hbm.at[pl.ds(ebase + jc * _DCH, _DCH)], idxbuf)

      @pl.loop(0, _DCH // _L)
      def _(i):
        plsc.addupdate_scatter(degloc, [idxbuf[pl.ds(i * _L, _L)]], ones16)

    pltpu.sync_copy(degloc, degp_hbm.at[w])

    @pl.loop(0, _TCH - 1, step=2)
    def _(j):
      tgt_finish(j, sv0, dv0, ar0, sem0)

      @pl.when(j + 2 < _TCH)
      def _():
        tgt_start(j + 2, sv0, dv0, ar0, sem0)

      tgt_finish(j + 1, sv1, dv1, ar1, sem1)

      @pl.when(j + 3 < _TCH)
      def _():
        tgt_start(j + 3, sv1, dv1, ar1, sem1)

    tgt_finish(_TCH - 1, sv0, dv0, ar0, sem0)

  @functools.partial(
      pl.kernel,
      out_type=jax.ShapeDtypeStruct((_NC, _NP, _H), _f32),
      mesh=mesh,
      compiler_params=pltpu.CompilerParams(needs_layout_passes=False, use_tc_tiling_on_sc=False),
      scratch_types=[
          pltpu.VMEM((_ECH, _CH), _i32),
          pltpu.VMEM((_ECH, _CH), _i32),
          pltpu.VMEM((_CH, _H), _f32),
          pltpu.VMEM((_CH, _H), _f32),
          pltpu.VMEM_SHARED((_NP, _H), _f32),
          pltpu.SemaphoreType.DMA,
          pltpu.SemaphoreType.DMA,
      ],
  )
  def sc_scatter(src2_hbm, dst2_hbm, g_hbm, z_hbm, out_hbm,
                 idxs2, idxd2, rows0, rows1, acc, sem0, sem1):
    c = lax.axis_index("c")
    s = lax.axis_index("s")
    row0 = s * _ROWS_T

    chunk0 = (c * _NS + s) * _ECH
    pltpu.sync_copy(src2_hbm.at[pl.ds(chunk0, _ECH)], idxs2)
    pltpu.sync_copy(dst2_hbm.at[pl.ds(chunk0, _ECH)], idxd2)

    @pl.when(c == 0)
    def _():
      pltpu.sync_copy(g_hbm.at[pl.ds(row0, _ROWS_T)],
                      acc.at[pl.ds(row0, _ROWS_T)])

    @pl.when(c == 1)
    def _():
      pltpu.sync_copy(z_hbm.at[pl.ds(row0, _ROWS_T)],
                      acc.at[pl.ds(row0, _ROWS_T)])

    pltpu.async_copy(g_hbm.at[idxs2.at[0]], rows0, sem0)
    pltpu.async_copy(g_hbm.at[idxs2.at[1]], rows1, sem1)
    plsc.subcore_barrier()

    @pl.loop(0, _ECH, step=2)
    def _(j):
      pltpu.make_async_copy(g_hbm.at[idxs2.at[0]], rows0, sem0).wait()
      pltpu.sync_copy(rows0, acc.at[idxd2.at[j]], add=True)

      @pl.when(j + 2 < _ECH)
      def _():
        pltpu.async_copy(g_hbm.at[idxs2.at[j + 2]], rows0, sem0)

      pltpu.make_async_copy(g_hbm.at[idxs2.at[1]], rows1, sem1).wait()
      pltpu.sync_copy(rows1, acc.at[idxd2.at[j + 1]], add=True)

      @pl.when(j + 3 < _ECH)
      def _():
        pltpu.async_copy(g_hbm.at[idxs2.at[j + 3]], rows1, sem1)

    plsc.subcore_barrier()
    pltpu.sync_copy(acc.at[pl.ds(row0, _ROWS_T)],
                    out_hbm.at[c, pl.ds(row0, _ROWS_T)])

  @functools.partial(
      pl.kernel,
      out_type=(
          jax.ShapeDtypeStruct((_TP, _H), _f32),
          jax.ShapeDtypeStruct((_TP, _H), _f32),
      ),
      mesh=mesh,
      compiler_params=pltpu.CompilerParams(needs_layout_passes=False, use_tc_tiling_on_sc=False),
      scratch_types=[
          pltpu.VMEM((_TCH, _CH), _i32),
          pltpu.VMEM((_TCH, _CH), _i32),
          pltpu.VMEM((_CH, _H), _f32),
          pltpu.VMEM((_CH, _H), _f32),
          pltpu.VMEM((_CH, _H), _f32),
          pltpu.VMEM((_CH, _H), _f32),
          pltpu.SemaphoreType.DMA,
          pltpu.SemaphoreType.DMA,
      ],
  )
  def sc_pq(sidx2_hbm, didx2_hbm, p_hbm, q_hbm, pg_hbm, qg_hbm,
            sv2, dv2, prow0, qrow0, prow1, qrow1, sem0, sem1):
    c = lax.axis_index("c")
    s = lax.axis_index("s")
    w = s * _NC + c
    tbase = w * _TPT
    tchunk0 = w * _TCH
    pltpu.sync_copy(sidx2_hbm.at[pl.ds(tchunk0, _TCH)], sv2)
    pltpu.sync_copy(didx2_hbm.at[pl.ds(tchunk0, _TCH)], dv2)

    def pq_start(j, prow, qrow, sem):
      pltpu.async_copy(p_hbm.at[sv2.at[j]], prow, sem)
      pltpu.async_copy(q_hbm.at[dv2.at[j]], qrow, sem)

    def pq_finish(j, prow, qrow, sem):
      t0 = tbase + j * _CH
      pltpu.make_async_copy(p_hbm.at[sv2.at[0]], prow, sem).wait()
      pltpu.make_async_copy(q_hbm.at[dv2.at[0]], qrow, sem).wait()
      pltpu.sync_copy(prow, pg_hbm.at[pl.ds(t0, _CH)])
      pltpu.sync_copy(qrow, qg_hbm.at[pl.ds(t0, _CH)])

    pq_start(0, prow0, qrow0, sem0)
    pq_start(1, prow1, qrow1, sem1)

    @pl.loop(0, _TCH - 1, step=2)
    def _(j):
      pq_finish(j, prow0, qrow0, sem0)

      @pl.when(j + 2 < _TCH)
      def _():
        pq_start(j + 2, prow0, qrow0, sem0)

      pq_finish(j + 1, prow1, qrow1, sem1)

      @pl.when(j + 3 < _TCH)
      def _():
        pq_start(j + 3, prow1, qrow1, sem1)

    pq_finish(_TCH - 1, prow0, qrow0, sem0)

  return sc_prep, sc_scatter, sc_pq


def _tc1_body(degp_ref, x_ref, w_ref, dinv_ref, g_ref):
  deg = jnp.sum(degp_ref[...], axis=1, keepdims=True) + 1.0
  dinv = lax.rsqrt(deg)
  dinv_ref[...] = dinv
  pre = jnp.dot(x_ref[...], w_ref[...], preferred_element_type=_f32)
  g_ref[...] = pre * dinv


_tc1 = pl.pallas_call(
    _tc1_body,
    out_shape=(
        jax.ShapeDtypeStruct((_NP, 1), _f32),
        jax.ShapeDtypeStruct((_NP, _H), _f32),
    ),
)


def _comb_body(p_ref, dinv_ref, b_ref, w_ref, g_ref):
  dinv = dinv_ref[...]
  h = jnp.maximum((p_ref[0] + p_ref[1]) * dinv + b_ref[...], 0.0)
  pre = jnp.dot(h, w_ref[...], preferred_element_type=_f32)
  g_ref[...] = pre * dinv


_comb = pl.pallas_call(
    _comb_body,
    out_shape=jax.ShapeDtypeStruct((_NP, _H), _f32),
)


def _comb3_body(p_ref, dinv_ref, b_ref, wa_ref, wb_ref, pout_ref, qout_ref):
  dinv = dinv_ref[...]
  h = jnp.maximum((p_ref[0] + p_ref[1]) * dinv + b_ref[...], 0.0)
  pout_ref[...] = jnp.dot(h, wa_ref[...], preferred_element_type=_f32)
  qout_ref[...] = jnp.dot(h, wb_ref[...], preferred_element_type=_f32)


_comb3 = pl.pallas_call(
    _comb3_body,
    out_shape=(
        jax.ShapeDtypeStruct((_NP, _H), _f32),
        jax.ShapeDtypeStruct((_NP, _H), _f32),
    ),
)


def _mlp_body(pg_ref, qg_ref, ag_ref, m1c_ref, m1b_ref, m2w_ref, m2b_ref,
              m3w_ref, m3b_ref, o_ref):
  e1 = (pg_ref[...] + qg_ref[...]
        + jnp.dot(ag_ref[...], m1c_ref[...], preferred_element_type=_f32)
        + m1b_ref[...])
  e1 = jnp.maximum(e1, 0.0)
  e2 = jnp.maximum(
      jnp.dot(e1, m2w_ref[...], preferred_element_type=_f32) + m2b_ref[...],
      0.0)
  z = jnp.sum(e2 * m3w_ref[...], axis=1, keepdims=True) + m3b_ref[...]
  o_ref[...] = 1.0 / (1.0 + jnp.exp(-z))


_MLP_GRID = 16
_MR = _TP // _MLP_GRID

_mlp = pl.pallas_call(
    _mlp_body,
    grid=(_MLP_GRID,),
    in_specs=[
        pl.BlockSpec((_MR, _H), lambda i: (i, 0)),
        pl.BlockSpec((_MR, _H), lambda i: (i, 0)),
        pl.BlockSpec((_MR, _DE), lambda i: (i, 0)),
        pl.BlockSpec((_DE, _H), lambda i: (0, 0)),
        pl.BlockSpec((1, _H), lambda i: (0, 0)),
        pl.BlockSpec((_H, _H // 2), lambda i: (0, 0)),
        pl.BlockSpec((1, _H // 2), lambda i: (0, 0)),
        pl.BlockSpec((1, _H // 2), lambda i: (0, 0)),
        pl.BlockSpec((1, 1), lambda i: (0, 0)),
    ],
    out_specs=pl.BlockSpec((_MR, 1), lambda i: (i, 0)),
    out_shape=jax.ShapeDtypeStruct((_TP, 1), _f32),
)


def kernel(x, edge_index, edge_attr, target_edges,
           W1, b1, W2, b2, W3, b3, M1w, M1b, M2w, M2b, M3w, M3b):
  sc_prep, sc_scatter, sc_pq = _sc_kernels()
  src = edge_index[0]
  dst = edge_index[1]
  pad_e = jnp.full((_EP - _E,), _NP - 1, dtype=_i32)
  src_p = jnp.concatenate([src, pad_e])
  dst_p = jnp.concatenate([dst, pad_e])
  te_p = jnp.concatenate([target_edges, jnp.zeros((_TP - _T,), _i32)])
  x_p = jnp.pad(x, ((0, _NP - _N), (0, 0)))
  zeros_nh = jnp.zeros((_NP, _H), _f32)

  src2 = src_p.reshape(_EP // _CH, _CH)
  dst2 = dst_p.reshape(_EP // _CH, _CH)

  degp, sidx, didx, ag = sc_prep(dst_p, te_p.reshape(_TP // _CH, _CH),
                                 src, dst, edge_attr)
  dinv, g1 = _tc1(degp.T, x_p, W1)
  p = sc_scatter(src2, dst2, g1, zeros_nh)
  g2 = _comb(p, dinv, b1.reshape(1, _H), W2)
  p = sc_scatter(src2, dst2, g2, zeros_nh)
  g3 = _comb(p, dinv, b2.reshape(1, _H), W3)
  p = sc_scatter(src2, dst2, g3, zeros_nh)
  P, Q = _comb3(p, dinv, b3.reshape(1, _H), M1w[:_H], M1w[_H:2 * _H])
  pg, qg = sc_pq(sidx.reshape(_TP // _CH, _CH),
                 didx.reshape(_TP // _CH, _CH), P, Q)
  out = _mlp(pg, qg, ag, M1w[2 * _H:], M1b.reshape(1, _H),
             M2w, M2b.reshape(1, _H // 2), M3w.reshape(1, _H // 2),
             M3b.reshape(1, 1))
  return out[:_T, 0]


# speedup vs baseline: 17.2460x; 1.6037x over previous
import functools

import jax
import jax.numpy as jnp
from jax import lax
from jax.experimental import pallas as pl
from jax.experimental.pallas import tpu as pltpu
from jax.experimental.pallas import tpu_sc as plsc

_N = 10000
_E = 320000
_T = 100000
_F = 128
_DE = 16
_H = 64

_NC, _NS, _L = 2, 16, 16
_NW = _NC * _NS

_NP = 10016
_ROWS_T = _NP // _NS
_CH = 128
_ECH = 80
_EPT = _CH * _ECH
_EP = _EPT * _NW
_TCH = 25
_TPT = _CH * _TCH
_TP = _TPT * _NW
_DCH = 2048

_f32 = jnp.float32
_i32 = jnp.int32


@functools.cache
def _sc_kernels():
  mesh = plsc.VectorSubcoreMesh(core_axis_name="c", subcore_axis_name="s",
                                num_cores=_NC, num_subcores=_NS)

  @functools.partial(
      pl.kernel,
      out_type=(
          jax.ShapeDtypeStruct((_NW, _NP), _f32),
          jax.ShapeDtypeStruct((_TP,), _i32),
          jax.ShapeDtypeStruct((_TP,), _i32),
          jax.ShapeDtypeStruct((_TP, _DE), _f32),
      ),
      mesh=mesh,
      compiler_params=pltpu.CompilerParams(needs_layout_passes=False, use_tc_tiling_on_sc=False),
      scratch_types=[
          pltpu.VMEM((_NP,), _f32),
          pltpu.VMEM((_DCH,), _i32),
          pltpu.VMEM((_TCH, _CH), _i32),
          pltpu.VMEM((_CH,), _i32),
          pltpu.VMEM((_CH,), _i32),
          pltpu.VMEM((_CH, _DE), _f32),
          pltpu.VMEM((_CH,), _i32),
          pltpu.VMEM((_CH,), _i32),
          pltpu.VMEM((_CH, _DE), _f32),
          pltpu.SemaphoreType.DMA,
          pltpu.SemaphoreType.DMA,
      ],
  )
  def sc_prep(dst_hbm, te2_hbm, ei0_hbm, ei1_hbm, ea_hbm,
              degp_hbm, sidx_hbm, didx_hbm, ag_hbm,
              degloc, idxbuf, te2, sv0, dv0, ar0, sv1, dv1, ar1,
              sem0, sem1):
    c = lax.axis_index("c")
    s = lax.axis_index("s")
    w = s * _NC + c
    ones16 = jnp.full((_L,), 1.0, _f32)
    zer16 = jnp.zeros((_L,), _f32)

    tchunk0 = w * _TCH
    pltpu.sync_copy(te2_hbm.at[pl.ds(tchunk0, _TCH)], te2)

    def tgt_start(j, sv, dv, ar, sem):
      pltpu.async_copy(ei0_hbm.at[te2.at[j]], sv, sem)
      pltpu.async_copy(ei1_hbm.at[te2.at[j]], dv, sem)
      pltpu.async_copy(ea_hbm.at[te2.at[j]], ar, sem)

    def tgt_finish(j, sv, dv, ar, sem):
      t0 = w * _TPT + j * _CH
      pltpu.make_async_copy(ei0_hbm.at[te2.at[0]], sv, sem).wait()
      pltpu.make_async_copy(ei1_hbm.at[te2.at[0]], dv, sem).wait()
      pltpu.make_async_copy(ea_hbm.at[te2.at[0]], ar, sem).wait()
      pltpu.sync_copy(sv, sidx_hbm.at[pl.ds(t0, _CH)])
      pltpu.sync_copy(dv, didx_hbm.at[pl.ds(t0, _CH)])
      pltpu.sync_copy(ar, ag_hbm.at[pl.ds(t0, _CH)])

    tgt_start(0, sv0, dv0, ar0, sem0)
    tgt_start(1, sv1, dv1, ar1, sem1)

    @pl.loop(0, _NP // _L)
    def _(i):
      degloc[pl.ds(i * _L, _L)] = zer16

    ebase = w * _EPT

    @pl.loop(0, _EPT // _DCH)
    def _(jc):
      pltpu.sync_copy(dst_hbm.at[pl.ds(ebase + jc * _DCH, _DCH)], idxbuf)

      @pl.loop(0, _DCH // _L)
      def _(i):
        plsc.addupdate_scatter(degloc, [idxbuf[pl.ds(i * _L, _L)]], ones16)

    pltpu.sync_copy(degloc, degp_hbm.at[w])

    @pl.loop(0, _TCH - 1, step=2)
    def _(j):
      tgt_finish(j, sv0, dv0, ar0, sem0)

      @pl.when(j + 2 < _TCH)
      def _():
        tgt_start(j + 2, sv0, dv0, ar0, sem0)

      tgt_finish(j + 1, sv1, dv1, ar1, sem1)

      @pl.when(j + 3 < _TCH)
      def _():
        tgt_start(j + 3, sv1, dv1, ar1, sem1)

    tgt_finish(_TCH - 1, sv0, dv0, ar0, sem0)

  @functools.partial(
      pl.kernel,
      out_type=jax.ShapeDtypeStruct((_NC, _NP, _H), _f32),
      mesh=mesh,
      compiler_params=pltpu.CompilerParams(needs_layout_passes=False, use_tc_tiling_on_sc=False),
      scratch_types=[
          pltpu.VMEM((_ECH, _CH), _i32),
          pltpu.VMEM((_ECH, _CH), _i32),
          pltpu.VMEM((_CH, _H), _f32),
          pltpu.VMEM((_CH, _H), _f32),
          pltpu.VMEM_SHARED((_NP, _H), _f32),
          pltpu.SemaphoreType.DMA,
          pltpu.SemaphoreType.DMA,
      ],
  )
  def sc_scatter(src2_hbm, dst2_hbm, g_hbm, z_hbm, out_hbm,
                 idxs2, idxd2, rows0, rows1, acc, sem0, sem1):
    c = lax.axis_index("c")
    s = lax.axis_index("s")
    row0 = s * _ROWS_T

    chunk0 = (c * _NS + s) * _ECH
    pltpu.sync_copy(src2_hbm.at[pl.ds(chunk0, _ECH)], idxs2)
    pltpu.sync_copy(dst2_hbm.at[pl.ds(chunk0, _ECH)], idxd2)

    @pl.when(c == 0)
    def _():
      pltpu.sync_copy(g_hbm.at[pl.ds(row0, _ROWS_T)],
                      acc.at[pl.ds(row0, _ROWS_T)])

    @pl.when(c == 1)
    def _():
      pltpu.sync_copy(z_hbm.at[pl.ds(row0, _ROWS_T)],
                      acc.at[pl.ds(row0, _ROWS_T)])

    pltpu.async_copy(g_hbm.at[idxs2.at[0]], rows0, sem0)
    pltpu.async_copy(g_hbm.at[idxs2.at[1]], rows1, sem1)
    plsc.subcore_barrier()

    @pl.loop(0, _ECH, step=2)
    def _(j):
      pltpu.make_async_copy(g_hbm.at[idxs2.at[0]], rows0, sem0).wait()
      pltpu.sync_copy(rows0, acc.at[idxd2.at[j]], add=True)

      @pl.when(j + 2 < _ECH)
      def _():
        pltpu.async_copy(g_hbm.at[idxs2.at[j + 2]], rows0, sem0)

      pltpu.make_async_copy(g_hbm.at[idxs2.at[1]], rows1, sem1).wait()
      pltpu.sync_copy(rows1, acc.at[idxd2.at[j + 1]], add=True)

      @pl.when(j + 3 < _ECH)
      def _():
        pltpu.async_copy(g_hbm.at[idxs2.at[j + 3]], rows1, sem1)

    plsc.subcore_barrier()
    pltpu.sync_copy(acc.at[pl.ds(row0, _ROWS_T)],
                    out_hbm.at[c, pl.ds(row0, _ROWS_T)])

  @functools.partial(
      pl.kernel,
      out_type=(
          jax.ShapeDtypeStruct((_TP, _H), _f32),
          jax.ShapeDtypeStruct((_TP, _H), _f32),
      ),
      mesh=mesh,
      compiler_params=pltpu.CompilerParams(needs_layout_passes=False, use_tc_tiling_on_sc=False),
      scratch_types=[
          pltpu.VMEM((_TCH, _CH), _i32),
          pltpu.VMEM((_TCH, _CH), _i32),
          pltpu.VMEM((_CH, _H), _f32),
          pltpu.VMEM((_CH, _H), _f32),
          pltpu.VMEM((_CH, _H), _f32),
          pltpu.VMEM((_CH, _H), _f32),
          pltpu.SemaphoreType.DMA,
          pltpu.SemaphoreType.DMA,
      ],
  )
  def sc_pq(sidx2_hbm, didx2_hbm, p_hbm, q_hbm, pg_hbm, qg_hbm,
            sv2, dv2, prow0, qrow0, prow1, qrow1, sem0, sem1):
    c = lax.axis_index("c")
    s = lax.axis_index("s")
    w = s * _NC + c
    tbase = w * _TPT
    tchunk0 = w * _TCH
    pltpu.sync_copy(sidx2_hbm.at[pl.ds(tchunk0, _TCH)], sv2)
    pltpu.sync_copy(didx2_hbm.at[pl.ds(tchunk0, _TCH)], dv2)

    def pq_start(j, prow, qrow, sem):
      pltpu.async_copy(p_hbm.at[sv2.at[j]], prow, sem)
      pltpu.async_copy(q_hbm.at[dv2.at[j]], qrow, sem)

    def pq_finish(j, prow, qrow, sem):
      t0 = tbase + j * _CH
      pltpu.make_async_copy(p_hbm.at[sv2.at[0]], prow, sem).wait()
      pltpu.make_async_copy(q_hbm.at[dv2.at[0]], qrow, sem).wait()
      pltpu.sync_copy(prow, pg_hbm.at[pl.ds(t0, _CH)])
      pltpu.sync_copy(qrow, qg_hbm.at[pl.ds(t0, _CH)])

    pq_start(0, prow0, qrow0, sem0)
    pq_start(1, prow1, qrow1, sem1)

    @pl.loop(0, _TCH - 1, step=2)
    def _(j):
      pq_finish(j, prow0, qrow0, sem0)

      @pl.when(j + 2 < _TCH)
      def _():
        pq_start(j + 2, prow0, qrow0, sem0)

      pq_finish(j + 1, prow1, qrow1, sem1)

      @pl.when(j + 3 < _TCH)
      def _():
        pq_start(j + 3, prow1, qrow1, sem1)

    pq_finish(_TCH - 1, prow0, qrow0, sem0)

  return sc_prep, sc_scatter, sc_pq


def _tc1_body(degp_ref, x_ref, w_ref, dinv_ref, g_ref):
  deg = jnp.sum(degp_ref[...], axis=1, keepdims=True) + 1.0
  dinv = lax.rsqrt(deg)
  dinv_ref[...] = dinv
  pre = jnp.dot(x_ref[...], w_ref[...], preferred_element_type=_f32)
  g_ref[...] = pre * dinv


_tc1 = pl.pallas_call(
    _tc1_body,
    out_shape=(
        jax.ShapeDtypeStruct((_NP, 1), _f32),
        jax.ShapeDtypeStruct((_NP, _H), _f32),
    ),
)


def _comb_body(p_ref, dinv_ref, b_ref, w_ref, g_ref):
  dinv = dinv_ref[...]
  h = jnp.maximum((p_ref[0] + p_ref[1]) * dinv + b_ref[...], 0.0)
  pre = jnp.dot(h, w_ref[...], preferred_element_type=_f32)
  g_ref[...] = pre * dinv


_comb = pl.pallas_call(
    _comb_body,
    out_shape=jax.ShapeDtypeStruct((_NP, _H), _f32),
)


def _comb3_body(p_ref, dinv_ref, b_ref, wa_ref, wb_ref, pout_ref, qout_ref):
  dinv = dinv_ref[...]
  h = jnp.maximum((p_ref[0] + p_ref[1]) * dinv + b_ref[...], 0.0)
  pout_ref[...] = jnp.dot(h, wa_ref[...], preferred_element_type=_f32)
  qout_ref[...] = jnp.dot(h, wb_ref[...], preferred_element_type=_f32)


_comb3 = pl.pallas_call(
    _comb3_body,
    out_shape=(
        jax.ShapeDtypeStruct((_NP, _H), _f32),
        jax.ShapeDtypeStruct((_NP, _H), _f32),
    ),
)


def _mlp_body(pg_ref, qg_ref, ag_ref, m1c_ref, m1b_ref, m2w_ref, m2b_ref,
              m3w_ref, m3b_ref, o_ref):
  e1 = (pg_ref[...] + qg_ref[...]
        + jnp.dot(ag_ref[...], m1c_ref[...], preferred_element_type=_f32)
        + m1b_ref[...])
  e1 = jnp.maximum(e1, 0.0)
  e2 = jnp.maximum(
      jnp.dot(e1, m2w_ref[...], preferred_element_type=_f32) + m2b_ref[...],
      0.0)
  z = jnp.sum(e2 * m3w_ref[...], axis=1, keepdims=True) + m3b_ref[...]
  o_ref[...] = 1.0 / (1.0 + jnp.exp(-z))


_MLP_GRID = 16
_MR = _TP // _MLP_GRID

_mlp = pl.pallas_call(
    _mlp_body,
    grid=(_MLP_GRID,),
    in_specs=[
        pl.BlockSpec((_MR, _H), lambda i: (i, 0)),
        pl.BlockSpec((_MR, _H), lambda i: (i, 0)),
        pl.BlockSpec((_MR, _DE), lambda i: (i, 0)),
        pl.BlockSpec((_DE, _H), lambda i: (0, 0)),
        pl.BlockSpec((1, _H), lambda i: (0, 0)),
        pl.BlockSpec((_H, _H // 2), lambda i: (0, 0)),
        pl.BlockSpec((1, _H // 2), lambda i: (0, 0)),
        pl.BlockSpec((1, _H // 2), lambda i: (0, 0)),
        pl.BlockSpec((1, 1), lambda i: (0, 0)),
    ],
    out_specs=pl.BlockSpec((_MR, 1), lambda i: (i, 0)),
    out_shape=jax.ShapeDtypeStruct((_TP, 1), _f32),
)


def kernel(x, edge_index, edge_attr, target_edges,
           W1, b1, W2, b2, W3, b3, M1w, M1b, M2w, M2b, M3w, M3b):
  sc_prep, sc_scatter, sc_pq = _sc_kernels()
  src = edge_index[0]
  dst = edge_index[1]
  pad_e = _N + (jnp.arange(_EP - _E, dtype=_i32) % (_NP - _N))
  src_p = jnp.concatenate([src, pad_e])
  dst_p = jnp.concatenate([dst, pad_e])
  te_p = jnp.concatenate([target_edges, jnp.arange(_TP - _T, dtype=_i32)])
  x_p = jnp.pad(x, ((0, _NP - _N), (0, 0)))
  zeros_nh = jnp.zeros((_NP, _H), _f32)

  src2 = src_p.reshape(_EP // _CH, _CH)
  dst2 = dst_p.reshape(_EP // _CH, _CH)

  degp, sidx, didx, ag = sc_prep(dst_p, te_p.reshape(_TP // _CH, _CH),
                                 src, dst, edge_attr)
  dinv, g1 = _tc1(degp.T, x_p, W1)
  p = sc_scatter(src2, dst2, g1, zeros_nh)
  g2 = _comb(p, dinv, b1.reshape(1, _H), W2)
  p = sc_scatter(src2, dst2, g2, zeros_nh)
  g3 = _comb(p, dinv, b2.reshape(1, _H), W3)
  p = sc_scatter(src2, dst2, g3, zeros_nh)
  P, Q = _comb3(p, dinv, b3.reshape(1, _H), M1w[:_H], M1w[_H:2 * _H])
  pg, qg = sc_pq(sidx.reshape(_TP // _CH, _CH),
                 didx.reshape(_TP // _CH, _CH), P, Q)
  out = _mlp(pg, qg, ag, M1w[2 * _H:], M1b.reshape(1, _H),
             M2w, M2b.reshape(1, _H // 2), M3w.reshape(1, _H // 2),
             M3b.reshape(1, 1))
  return out[:_T, 0]


# speedup vs baseline: 17.6391x; 1.0228x over previous
import functools

import jax
import jax.numpy as jnp
from jax import lax
from jax.experimental import pallas as pl
from jax.experimental.pallas import tpu as pltpu
from jax.experimental.pallas import tpu_sc as plsc

_N = 10000
_E = 320000
_T = 100000
_F = 128
_DE = 16
_H = 64

_NC, _NS, _L = 2, 16, 16
_NW = _NC * _NS

_NP = 10016
_ROWS_T = _NP // _NS
_CH = 128
_ECH = 80
_EPT = _CH * _ECH
_EP = _EPT * _NW
_TCH = 25
_TPT = _CH * _TCH
_TP = _TPT * _NW
_DCH = 2048

_f32 = jnp.float32
_i32 = jnp.int32


@functools.cache
def _sc_kernels():
  mesh = plsc.VectorSubcoreMesh(core_axis_name="c", subcore_axis_name="s",
                                num_cores=_NC, num_subcores=_NS)

  @functools.partial(
      pl.kernel,
      out_type=(
          jax.ShapeDtypeStruct((_NW, _NP), _f32),
          jax.ShapeDtypeStruct((_TP,), _i32),
          jax.ShapeDtypeStruct((_TP,), _i32),
          jax.ShapeDtypeStruct((_TP, _DE), _f32),
      ),
      mesh=mesh,
      compiler_params=pltpu.CompilerParams(needs_layout_passes=False, use_tc_tiling_on_sc=False),
      scratch_types=[
          pltpu.VMEM((_NP,), _f32),
          pltpu.VMEM((_DCH,), _i32),
          pltpu.VMEM((_TCH, _CH), _i32),
          pltpu.VMEM((_CH,), _i32),
          pltpu.VMEM((_CH,), _i32),
          pltpu.VMEM((_CH, _DE), _f32),
          pltpu.VMEM((_CH,), _i32),
          pltpu.VMEM((_CH,), _i32),
          pltpu.VMEM((_CH, _DE), _f32),
          pltpu.SemaphoreType.DMA,
          pltpu.SemaphoreType.DMA,
      ],
  )
  def sc_prep(dst_hbm, te2_hbm, ei0_hbm, ei1_hbm, ea_hbm,
              degp_hbm, sidx_hbm, didx_hbm, ag_hbm,
              degloc, idxbuf, te2, sv0, dv0, ar0, sv1, dv1, ar1,
              sem0, sem1):
    c = lax.axis_index("c")
    s = lax.axis_index("s")
    w = s * _NC + c
    ones16 = jnp.full((_L,), 1.0, _f32)
    zer16 = jnp.zeros((_L,), _f32)

    tchunk0 = w * _TCH
    pltpu.sync_copy(te2_hbm.at[pl.ds(tchunk0, _TCH)], te2)

    def tgt_start(j, sv, dv, ar, sem):
      pltpu.async_copy(ei0_hbm.at[te2.at[j]], sv, sem)
      pltpu.async_copy(ei1_hbm.at[te2.at[j]], dv, sem)
      pltpu.async_copy(ea_hbm.at[te2.at[j]], ar, sem)

    def tgt_finish(j, sv, dv, ar, sem):
      t0 = w * _TPT + j * _CH
      pltpu.make_async_copy(ei0_hbm.at[te2.at[0]], sv, sem).wait()
      pltpu.make_async_copy(ei1_hbm.at[te2.at[0]], dv, sem).wait()
      pltpu.make_async_copy(ea_hbm.at[te2.at[0]], ar, sem).wait()
      pltpu.sync_copy(sv, sidx_hbm.at[pl.ds(t0, _CH)])
      pltpu.sync_copy(dv, didx_hbm.at[pl.ds(t0, _CH)])
      pltpu.sync_copy(ar, ag_hbm.at[pl.ds(t0, _CH)])

    tgt_start(0, sv0, dv0, ar0, sem0)
    tgt_start(1, sv1, dv1, ar1, sem1)

    @pl.loop(0, _NP // _L)
    def _(i):
      degloc[pl.ds(i * _L, _L)] = zer16

    ebase = w * _EPT

    @pl.loop(0, _EPT // _DCH)
    def _(jc):
      pltpu.sync_copy(dst_hbm.at[pl.ds(ebase + jc * _DCH, _DCH)], idxbuf)

      @pl.loop(0, _DCH // _L)
      def _(i):
        plsc.addupdate_scatter(degloc, [idxbuf[pl.ds(i * _L, _L)]], ones16)

    pltpu.sync_copy(degloc, degp_hbm.at[w])

    @pl.loop(0, _TCH - 1, step=2)
    def _(j):
      tgt_finish(j, sv0, dv0, ar0, sem0)

      @pl.when(j + 2 < _TCH)
      def _():
        tgt_start(j + 2, sv0, dv0, ar0, sem0)

      tgt_finish(j + 1, sv1, dv1, ar1, sem1)

      @pl.when(j + 3 < _TCH)
      def _():
        tgt_start(j + 3, sv1, dv1, ar1, sem1)

    tgt_finish(_TCH - 1, sv0, dv0, ar0, sem0)

  @functools.partial(
      pl.kernel,
      out_type=jax.ShapeDtypeStruct((_NC, _NP, _H), _f32),
      mesh=mesh,
      compiler_params=pltpu.CompilerParams(needs_layout_passes=False, use_tc_tiling_on_sc=False),
      scratch_types=[
          pltpu.VMEM((_ECH, _CH), _i32),
          pltpu.VMEM((_ECH, _CH), _i32),
          pltpu.VMEM((_CH, _H), _f32),
          pltpu.VMEM((_CH, _H), _f32),
          pltpu.VMEM((_CH, _H), _f32),
          pltpu.VMEM((_CH, _H), _f32),
          pltpu.VMEM_SHARED((_NP, _H), _f32),
          pltpu.SemaphoreType.DMA,
          pltpu.SemaphoreType.DMA,
          pltpu.SemaphoreType.DMA,
          pltpu.SemaphoreType.DMA,
      ],
  )
  def sc_scatter(src2_hbm, dst2_hbm, g_hbm, z_hbm, out_hbm,
                 idxs2, idxd2, rows0, rows1, rows2, rows3, acc,
                 sem0, sem1, sem2, sem3):
    c = lax.axis_index("c")
    s = lax.axis_index("s")
    row0 = s * _ROWS_T

    chunk0 = (c * _NS + s) * _ECH
    pltpu.sync_copy(src2_hbm.at[pl.ds(chunk0, _ECH)], idxs2)
    pltpu.sync_copy(dst2_hbm.at[pl.ds(chunk0, _ECH)], idxd2)

    @pl.when(c == 0)
    def _():
      pltpu.sync_copy(g_hbm.at[pl.ds(row0, _ROWS_T)],
                      acc.at[pl.ds(row0, _ROWS_T)])

    @pl.when(c == 1)
    def _():
      pltpu.sync_copy(z_hbm.at[pl.ds(row0, _ROWS_T)],
                      acc.at[pl.ds(row0, _ROWS_T)])

    ring = ((rows0, sem0), (rows1, sem1), (rows2, sem2), (rows3, sem3))
    for k, (rows, sem) in enumerate(ring):
      pltpu.async_copy(g_hbm.at[idxs2.at[k]], rows, sem)
    plsc.subcore_barrier()

    @pl.loop(0, _ECH, step=4)
    def _(j):
      for k, (rows, sem) in enumerate(ring):
        pltpu.make_async_copy(g_hbm.at[idxs2.at[0]], rows, sem).wait()
        pltpu.sync_copy(rows, acc.at[idxd2.at[j + k]], add=True)

        @pl.when(j + 4 + k < _ECH)
        def _():
          pltpu.async_copy(g_hbm.at[idxs2.at[j + 4 + k]], rows, sem)

    plsc.subcore_barrier()
    pltpu.sync_copy(acc.at[pl.ds(row0, _ROWS_T)],
                    out_hbm.at[c, pl.ds(row0, _ROWS_T)])

  @functools.partial(
      pl.kernel,
      out_type=(
          jax.ShapeDtypeStruct((_TP, _H), _f32),
          jax.ShapeDtypeStruct((_TP, _H), _f32),
      ),
      mesh=mesh,
      compiler_params=pltpu.CompilerParams(needs_layout_passes=False, use_tc_tiling_on_sc=False),
      scratch_types=[
          pltpu.VMEM((_TCH, _CH), _i32),
          pltpu.VMEM((_TCH, _CH), _i32),
          pltpu.VMEM((_CH, _H), _f32),
          pltpu.VMEM((_CH, _H), _f32),
          pltpu.VMEM((_CH, _H), _f32),
          pltpu.VMEM((_CH, _H), _f32),
          pltpu.SemaphoreType.DMA,
          pltpu.SemaphoreType.DMA,
      ],
  )
  def sc_pq(sidx2_hbm, didx2_hbm, p_hbm, q_hbm, pg_hbm, qg_hbm,
            sv2, dv2, prow0, qrow0, prow1, qrow1, sem0, sem1):
    c = lax.axis_index("c")
    s = lax.axis_index("s")
    w = s * _NC + c
    tbase = w * _TPT
    tchunk0 = w * _TCH
    pltpu.sync_copy(sidx2_hbm.at[pl.ds(tchunk0, _TCH)], sv2)
    pltpu.sync_copy(didx2_hbm.at[pl.ds(tchunk0, _TCH)], dv2)

    def pq_start(j, prow, qrow, sem):
      pltpu.async_copy(p_hbm.at[sv2.at[j]], prow, sem)
      pltpu.async_copy(q_hbm.at[dv2.at[j]], qrow, sem)

    def pq_finish(j, prow, qrow, sem):
      t0 = tbase + j * _CH
      pltpu.make_async_copy(p_hbm.at[sv2.at[0]], prow, sem).wait()
      pltpu.make_async_copy(q_hbm.at[dv2.at[0]], qrow, sem).wait()
      pltpu.sync_copy(prow, pg_hbm.at[pl.ds(t0, _CH)])
      pltpu.sync_copy(qrow, qg_hbm.at[pl.ds(t0, _CH)])

    pq_start(0, prow0, qrow0, sem0)
    pq_start(1, prow1, qrow1, sem1)

    @pl.loop(0, _TCH - 1, step=2)
    def _(j):
      pq_finish(j, prow0, qrow0, sem0)

      @pl.when(j + 2 < _TCH)
      def _():
        pq_start(j + 2, prow0, qrow0, sem0)

      pq_finish(j + 1, prow1, qrow1, sem1)

      @pl.when(j + 3 < _TCH)
      def _():
        pq_start(j + 3, prow1, qrow1, sem1)

    pq_finish(_TCH - 1, prow0, qrow0, sem0)

  return sc_prep, sc_scatter, sc_pq


def _tc1_body(degp_ref, x_ref, w_ref, dinv_ref, g_ref):
  deg = jnp.sum(degp_ref[...], axis=1, keepdims=True) + 1.0
  dinv = lax.rsqrt(deg)
  dinv_ref[...] = dinv
  pre = jnp.dot(x_ref[...], w_ref[...], preferred_element_type=_f32)
  g_ref[...] = pre * dinv


_tc1 = pl.pallas_call(
    _tc1_body,
    out_shape=(
        jax.ShapeDtypeStruct((_NP, 1), _f32),
        jax.ShapeDtypeStruct((_NP, _H), _f32),
    ),
)


def _comb_body(p_ref, dinv_ref, b_ref, w_ref, g_ref):
  dinv = dinv_ref[...]
  h = jnp.maximum((p_ref[0] + p_ref[1]) * dinv + b_ref[...], 0.0)
  pre = jnp.dot(h, w_ref[...], preferred_element_type=_f32)
  g_ref[...] = pre * dinv


_comb = pl.pallas_call(
    _comb_body,
    out_shape=jax.ShapeDtypeStruct((_NP, _H), _f32),
)


def _comb3_body(p_ref, dinv_ref, b_ref, wa_ref, wb_ref, pout_ref, qout_ref):
  dinv = dinv_ref[...]
  h = jnp.maximum((p_ref[0] + p_ref[1]) * dinv + b_ref[...], 0.0)
  pout_ref[...] = jnp.dot(h, wa_ref[...], preferred_element_type=_f32)
  qout_ref[...] = jnp.dot(h, wb_ref[...], preferred_element_type=_f32)


_comb3 = pl.pallas_call(
    _comb3_body,
    out_shape=(
        jax.ShapeDtypeStruct((_NP, _H), _f32),
        jax.ShapeDtypeStruct((_NP, _H), _f32),
    ),
)


def _mlp_body(pg_ref, qg_ref, ag_ref, m1c_ref, m1b_ref, m2w_ref, m2b_ref,
              m3w_ref, m3b_ref, o_ref):
  e1 = (pg_ref[...] + qg_ref[...]
        + jnp.dot(ag_ref[...], m1c_ref[...], preferred_element_type=_f32)
        + m1b_ref[...])
  e1 = jnp.maximum(e1, 0.0)
  e2 = jnp.maximum(
      jnp.dot(e1, m2w_ref[...], preferred_element_type=_f32) + m2b_ref[...],
      0.0)
  z = jnp.sum(e2 * m3w_ref[...], axis=1, keepdims=True) + m3b_ref[...]
  o_ref[...] = 1.0 / (1.0 + jnp.exp(-z))


_MLP_GRID = 16
_MR = _TP // _MLP_GRID

_mlp = pl.pallas_call(
    _mlp_body,
    grid=(_MLP_GRID,),
    in_specs=[
        pl.BlockSpec((_MR, _H), lambda i: (i, 0)),
        pl.BlockSpec((_MR, _H), lambda i: (i, 0)),
        pl.BlockSpec((_MR, _DE), lambda i: (i, 0)),
        pl.BlockSpec((_DE, _H), lambda i: (0, 0)),
        pl.BlockSpec((1, _H), lambda i: (0, 0)),
        pl.BlockSpec((_H, _H // 2), lambda i: (0, 0)),
        pl.BlockSpec((1, _H // 2), lambda i: (0, 0)),
        pl.BlockSpec((1, _H // 2), lambda i: (0, 0)),
        pl.BlockSpec((1, 1), lambda i: (0, 0)),
    ],
    out_specs=pl.BlockSpec((_MR, 1), lambda i: (i, 0)),
    out_shape=jax.ShapeDtypeStruct((_TP, 1), _f32),
)


def kernel(x, edge_index, edge_attr, target_edges,
           W1, b1, W2, b2, W3, b3, M1w, M1b, M2w, M2b, M3w, M3b):
  sc_prep, sc_scatter, sc_pq = _sc_kernels()
  src = edge_index[0]
  dst = edge_index[1]
  pad_e = _N + (jnp.arange(_EP - _E, dtype=_i32) % (_NP - _N))
  src_p = jnp.concatenate([src, pad_e])
  dst_p = jnp.concatenate([dst, pad_e])
  te_p = jnp.concatenate([target_edges, jnp.arange(_TP - _T, dtype=_i32)])
  x_p = jnp.pad(x, ((0, _NP - _N), (0, 0)))
  zeros_nh = jnp.zeros((_NP, _H), _f32)

  src2 = src_p.reshape(_EP // _CH, _CH)
  dst2 = dst_p.reshape(_EP // _CH, _CH)

  degp, sidx, didx, ag = sc_prep(dst_p, te_p.reshape(_TP // _CH, _CH),
                                 src, dst, edge_attr)
  dinv, g1 = _tc1(degp.T, x_p, W1)
  p = sc_scatter(src2, dst2, g1, zeros_nh)
  g2 = _comb(p, dinv, b1.reshape(1, _H), W2)
  p = sc_scatter(src2, dst2, g2, zeros_nh)
  g3 = _comb(p, dinv, b2.reshape(1, _H), W3)
  p = sc_scatter(src2, dst2, g3, zeros_nh)
  P, Q = _comb3(p, dinv, b3.reshape(1, _H), M1w[:_H], M1w[_H:2 * _H])
  pg, qg = sc_pq(sidx.reshape(_TP // _CH, _CH),
                 didx.reshape(_TP // _CH, _CH), P, Q)
  out = _mlp(pg, qg, ag, M1w[2 * _H:], M1b.reshape(1, _H),
             M2w, M2b.reshape(1, _H // 2), M3w.reshape(1, _H // 2),
             M3b.reshape(1, 1))
  return out[:_T, 0]
